# Initial kernel scaffold; baseline (speedup 1.0000x reference)
import jax
import jax.numpy as jnp
from jax.experimental import pallas as pl


def kernel(detection_result, gt_xywh, gt_class):
    raise NotImplementedError("write your pallas kernel here")


# speedup vs baseline: 6.9238x; 6.9238x over previous
import functools

import jax
import jax.numpy as jnp
import numpy as np
from jax.experimental import pallas as pl
from jax.experimental.pallas import tpu as pltpu

_ANCHORS = np.array([1.3221, 1.73145, 3.19275, 4.00944, 5.05587,
                     8.09892, 9.47112, 4.84053, 11.2364, 10.0071], np.float32)
_GRID = 13
_NA = 5
_NC = 20
_B = 128
_CELLS = _GRID * _GRID
_CH = 125
_N_TOTAL = _B * _CH * _CELLS

_SCALE = np.ones((_CH, 1), np.float32)
for _a in range(_NA):
    _SCALE[_a * 25 + 2, 0] = _ANCHORS[2 * _a] / _GRID
    _SCALE[_a * 25 + 3, 0] = _ANCHORS[2 * _a + 1] / _GRID


def _loss_body(det_ref, gt_ref, cls_ref, scale_ref, out_ref):
    pid = pl.program_id(0)
    nprog = pl.num_programs(0)

    det = det_ref[...] * scale_ref[...][None, :, :]
    bblk = det.shape[0]

    gt = gt_ref[...]
    gtx, gty, gtw, gth = gt[..., 0], gt[..., 1], gt[..., 2], gt[..., 3]
    gx = jnp.clip((gtx * _GRID).astype(jnp.int32), 0, _GRID - 1)
    gy = jnp.clip((gty * _GRID).astype(jnp.int32), 0, _GRID - 1)
    cell = gx * _GRID + gy

    cell_iota = jax.lax.broadcasted_iota(jnp.int32, (bblk, 8, _CELLS), 2)
    oh = (cell[:, :, None] == cell_iota).astype(jnp.float32)
    feats = jax.lax.dot_general(
        oh, det, (((2,), (2,)), ((0,), (0,))),
        preferred_element_type=jnp.float32)

    cls = cls_ref[...]
    cls_iota = jax.lax.broadcasted_iota(jnp.int32, (bblk, 8, _NC), 2)
    oh_cls = (cls[:, :, None] == cls_iota).astype(jnp.float32)

    a1 = (gtw - gtx + 1.0) * (gth - gty + 1.0)
    best = jnp.zeros(cell.shape, jnp.int32)
    best_iou = None
    tds = []
    for a in range(_NA):
        f = feats[:, :, a * 25:a * 25 + 25]
        f0, f1, f2, f3, f4 = (f[..., k] for k in range(5))
        x1 = jnp.maximum(gtx, f0)
        y1 = jnp.maximum(gty, f1)
        x2 = jnp.minimum(gtw, f2)
        y2 = jnp.minimum(gth, f3)
        inter = (x2 - x1 + 1.0) * (y2 - y1 + 1.0)
        a2 = (f2 - f0 + 1.0) * (f3 - f1 + 1.0)
        iou = inter / (a1 + a2 - inter)
        if best_iou is None:
            best_iou = iou
        else:
            take = iou > best_iou
            best = jnp.where(take, a, best)
            best_iou = jnp.where(take, iou, best_iou)
        cls_term = jnp.sum(oh_cls * f[..., 5:], axis=-1)
        tds.append(5.0 * (gtx * f0 + gty * f1 + gtw * f2 + gth * f3)
                   + f4 + cls_term)

    td = tds[0]
    for a in range(1, _NA):
        td = jnp.where(best == a, tds[a], td)
    t2 = 25.0 * (gtx * gtx + gty * gty + gtw * gtw + gth * gth) + 2.0

    key = cell * _NA + best
    eq = key[:, :, None] == key[:, None, :]
    oi = jax.lax.broadcasted_iota(jnp.int32, (bblk, 8, 8), 1)
    oj = jax.lax.broadcasted_iota(jnp.int32, (bblk, 8, 8), 2)
    overwritten = jnp.any(eq & (oj > oi), axis=2)
    corr = jnp.where(overwritten, 0.0, t2 - 2.0 * td)

    partial = jnp.sum(det * det) + jnp.sum(corr)

    @pl.when(pid == 0)
    def _init():
        out_ref[...] = jnp.zeros((1, 1), jnp.float32)

    out_ref[...] += partial.reshape(1, 1)

    @pl.when(pid == nprog - 1)
    def _fin():
        out_ref[...] = out_ref[...] * (1.0 / _N_TOTAL)


@functools.partial(jax.jit, static_argnames=("interpret",))
def kernel(detection_result, gt_xywh, gt_class, interpret=False):
    det3 = detection_result.reshape(_B, _CH, _CELLS)
    bblk = 16
    out = pl.pallas_call(
        _loss_body,
        grid=(_B // bblk,),
        in_specs=[
            pl.BlockSpec((bblk, _CH, _CELLS), lambda i: (i, 0, 0)),
            pl.BlockSpec((bblk, 8, 4), lambda i: (i, 0, 0)),
            pl.BlockSpec((bblk, 8), lambda i: (i, 0)),
            pl.BlockSpec((_CH, 1), lambda i: (0, 0)),
        ],
        out_specs=pl.BlockSpec((1, 1), lambda i: (0, 0)),
        out_shape=jax.ShapeDtypeStruct((1, 1), jnp.float32),
        interpret=interpret,
    )(det3, gt_xywh, gt_class.astype(jnp.int32), jnp.asarray(_SCALE))
    return out[0, 0]
